# baseline (speedup 1.0000x reference)
import functools

import jax
import jax.numpy as jnp
from jax.experimental import pallas as pl
from jax.experimental.pallas import tpu as pltpu

_T = 16
_NCHUNK = 4


def _fused_kernel(m_ref, x_ref, eye_ref, o_ref, acc_ref, *, n_t):
    j = pl.program_id(1)

    @pl.when(j < _NCHUNK)
    def _():
        m = m_ref[...]
        for r in range(_T):
            valid = (j * _T + r) < n_t
            xa = jnp.where(valid, x_ref[0, r], 0.0)
            acc_ref[pl.ds(j, 1), :, r, :] = jax.lax.dot_general(
                m, xa,
                dimension_numbers=(((1,), (1,)), ((), ())),
                preferred_element_type=jnp.float32,
            )[None]

    @pl.when(j == _NCHUNK)
    def _():
        for g in range(o_ref.shape[1]):
            s = acc_ref[:, g].reshape(_NCHUNK * _T, acc_ref.shape[3])
            st = jax.lax.dot_general(
                s, eye_ref[...],
                dimension_numbers=(((0,), (0,)), ((), ())),
                preferred_element_type=jnp.float32,
            )
            o_ref[0, g] = st[:, :n_t].reshape(
                o_ref.shape[2], o_ref.shape[3], n_t)


def kernel(x, freq_masks):
    b, t, f, c1, c2 = x.shape
    g = freq_masks.shape[0]
    c = c1 * c2

    xr = x.transpose(0, 1, 3, 4, 2).reshape(b, t, c, f)

    counts = jnp.sum(freq_masks, axis=1, keepdims=True)
    sm = freq_masks / counts
    eye = jnp.eye(_NCHUNK * _T, dtype=jnp.float32)

    out5 = pl.pallas_call(
        functools.partial(_fused_kernel, n_t=t),
        out_shape=jax.ShapeDtypeStruct((b, g, c1, c2, t), jnp.float32),
        grid=(b, _NCHUNK + 1),
        in_specs=[
            pl.BlockSpec((g, f), lambda i, j: (0, 0)),
            pl.BlockSpec((1, _T, c, f),
                         lambda i, j: (i, jnp.minimum(j, _NCHUNK - 1), 0, 0)),
            pl.BlockSpec((_NCHUNK * _T, _NCHUNK * _T), lambda i, j: (0, 0)),
        ],
        out_specs=pl.BlockSpec((1, g, c1, c2, t),
                               lambda i, j: (i, 0, 0, 0, 0)),
        scratch_shapes=[pltpu.VMEM((_NCHUNK, g, _T, c), jnp.float32)],
        compiler_params=pltpu.CompilerParams(
            dimension_semantics=("parallel", "arbitrary"),
            vmem_limit_bytes=56 * 1024 * 1024,
        ),
        name="freq_band_avg",
    )(sm, xr, eye)

    return out5.transpose(0, 4, 1, 2, 3)


# speedup vs baseline: 1.1028x; 1.1028x over previous
import functools

import jax
import jax.numpy as jnp
from jax.experimental import pallas as pl
from jax.experimental.pallas import tpu as pltpu

_T = 8
_NCHUNK = 7
_FT = 128


def _fused_kernel(m_ref, x_ref, x2_ref, eye_ref, o_ref, acc_ref, *, n_t):
    j = pl.program_id(1)

    @pl.when(j < _NCHUNK)
    def _():
        m0 = m_ref[:, :_FT]
        m1 = m_ref[:, _FT:]
        for r in range(_T):
            valid = (j * _T + r) < n_t
            xa = jnp.where(valid, x_ref[0, r], 0.0)
            xb = jnp.where(valid, x2_ref[0, r, :, :1], 0.0)
            res = jax.lax.dot_general(
                m0, xa,
                dimension_numbers=(((1,), (1,)), ((), ())),
                preferred_element_type=jnp.float32,
            ) + m1 * xb.T
            acc_ref[pl.ds(j, 1), :, r, :] = res[None]

    @pl.when(j == _NCHUNK)
    def _():
        for g in range(o_ref.shape[1]):
            s = acc_ref[:, g].reshape(_NCHUNK * _T, acc_ref.shape[3])
            st = jax.lax.dot_general(
                s, eye_ref[...],
                dimension_numbers=(((0,), (0,)), ((), ())),
                preferred_element_type=jnp.float32,
            )
            o_ref[0, g] = st[:, :n_t].reshape(
                o_ref.shape[2], o_ref.shape[3], n_t)


def kernel(x, freq_masks):
    b, t, f, c1, c2 = x.shape
    g = freq_masks.shape[0]
    c = c1 * c2

    xr = x.transpose(0, 1, 3, 4, 2).reshape(b, t, c, f)

    counts = jnp.sum(freq_masks, axis=1, keepdims=True)
    sm = freq_masks / counts
    eye = jnp.eye(_NCHUNK * _T, dtype=jnp.float32)

    def xchunk(i, j):
        return (i, jnp.minimum(j, _NCHUNK - 1), 0, 0)

    def x2chunk(i, j):
        return (i, jnp.minimum(j, _NCHUNK - 1), 0, 1)

    out5 = pl.pallas_call(
        functools.partial(_fused_kernel, n_t=t),
        out_shape=jax.ShapeDtypeStruct((b, g, c1, c2, t), jnp.float32),
        grid=(b, _NCHUNK + 1),
        in_specs=[
            pl.BlockSpec((g, f), lambda i, j: (0, 0)),
            pl.BlockSpec((1, _T, c, _FT), xchunk),
            pl.BlockSpec((1, _T, c, _FT), x2chunk),
            pl.BlockSpec((_NCHUNK * _T, _NCHUNK * _T), lambda i, j: (0, 0)),
        ],
        out_specs=pl.BlockSpec((1, g, c1, c2, t),
                               lambda i, j: (i, 0, 0, 0, 0)),
        scratch_shapes=[pltpu.VMEM((_NCHUNK, g, _T, c), jnp.float32)],
        compiler_params=pltpu.CompilerParams(
            dimension_semantics=("parallel", "arbitrary"),
            vmem_limit_bytes=56 * 1024 * 1024,
        ),
        name="freq_band_avg",
    )(sm, xr, xr, eye)

    return out5.transpose(0, 4, 1, 2, 3)
